# baseline (speedup 1.0000x reference)
import functools

import jax
import jax.numpy as jnp
from jax import lax
from jax.experimental import pallas as pl
from jax.experimental.pallas import tpu as pltpu
from jax.experimental.pallas import tpu_sc as plsc

NC = 2
NS = 16
NW = NC * NS
BLK = 128
LANES = 16
RING = 2
SB = 8


def _fill_zeros_1d(ref, n):
    def body(i, _):
        ref[pl.ds(i * LANES, LANES)] = jnp.zeros((LANES,), ref.dtype)
        return 0
    lax.fori_loop(0, n // LANES, body, 0)


def _fill_zeros_2d(ref, rows, cols):
    def body(i, _):
        def inner(j, _):
            ref[i, pl.ds(j * LANES, LANES)] = jnp.zeros((LANES,), ref.dtype)
            return 0
        lax.fori_loop(0, cols // LANES, inner, 0)
        return 0
    lax.fori_loop(0, rows, body, 0)


def _reduce_parts_and_emit(c, s, rpt, triples):
    for acc, parts, _out, _ta, _tb in triples:
        pltpu.sync_copy(acc, parts.at[s])
    plsc.subcore_barrier()
    base = s * rpt
    for _acc, parts, out, ta, tb in triples:
        pltpu.sync_copy(parts.at[0, pl.ds(base, rpt)], ta)

        def tile_body(t, _):
            pltpu.sync_copy(parts.at[t, pl.ds(base, rpt)], tb)

            def add_body(i, _):
                ta[pl.ds(i * LANES, LANES)] = (
                    ta[pl.ds(i * LANES, LANES)] + tb[pl.ds(i * LANES, LANES)])
                return 0
            lax.fori_loop(0, rpt // LANES, add_body, 0)
            return 0
        lax.fori_loop(1, NS, tile_body, 0)
        pltpu.sync_copy(ta, out.at[c, pl.ds(base, rpt)])


def _make_deg_kernel(n_pad, e_pad):
    ept = e_pad // NW
    nblk = ept // BLK
    rpt = n_pad // NS
    mesh = plsc.VectorSubcoreMesh(core_axis_name="c", subcore_axis_name="s")

    @functools.partial(
        pl.kernel,
        out_type=[
            jax.ShapeDtypeStruct((NC, n_pad), jnp.float32),
            jax.ShapeDtypeStruct((NC, n_pad), jnp.float32),
        ],
        mesh=mesh,
        compiler_params=pltpu.CompilerParams(needs_layout_passes=False),
        scratch_types=[
            pltpu.VMEM((nblk, BLK), jnp.int32),
            pltpu.VMEM((nblk, BLK), jnp.int32),
            pltpu.VMEM((n_pad,), jnp.float32),
            pltpu.VMEM((n_pad,), jnp.float32),
            pltpu.VMEM((rpt,), jnp.float32),
            pltpu.VMEM((rpt,), jnp.float32),
            pltpu.VMEM_SHARED((NS, n_pad), jnp.float32),
            pltpu.VMEM_SHARED((NS, n_pad), jnp.float32),
        ],
    )
    def deg_kernel(src_hbm, dst_hbm, out_o, out_i, sidx, didx, acc_o, acc_i,
                   tmp_a, tmp_b, parts_o, parts_i):
        c = lax.axis_index("c")
        s = lax.axis_index("s")
        w = c * NS + s
        _fill_zeros_1d(acc_o, n_pad)
        _fill_zeros_1d(acc_i, n_pad)
        pltpu.sync_copy(src_hbm.at[pl.ds(w * nblk, nblk)], sidx)
        pltpu.sync_copy(dst_hbm.at[pl.ds(w * nblk, nblk)], didx)
        ones = jnp.ones((LANES,), jnp.float32)

        def body(k, _):
            b = k // (BLK // LANES)
            j = (k % (BLK // LANES)) * LANES
            si = sidx[b, pl.ds(j, LANES)]
            di = didx[b, pl.ds(j, LANES)]
            plsc.addupdate_scatter(acc_o, [si], ones)
            plsc.addupdate_scatter(acc_i, [di], ones)
            return 0
        lax.fori_loop(0, ept // LANES, body, 0)
        _reduce_parts_and_emit(c, s, rpt, [
            (acc_o, parts_o, out_o, tmp_a, tmp_b),
            (acc_i, parts_i, out_i, tmp_a, tmp_b),
        ])

    return deg_kernel


def _make_scalar_segsum_kernel(n_pad, e_pad, nv):
    ept = e_pad // NW
    nblk = ept // BLK
    rpt = n_pad // NS
    mesh = plsc.VectorSubcoreMesh(core_axis_name="c", subcore_axis_name="s")

    @functools.partial(
        pl.kernel,
        out_type=[jax.ShapeDtypeStruct((NC, n_pad), jnp.float32)] * nv,
        mesh=mesh,
        compiler_params=pltpu.CompilerParams(needs_layout_passes=False),
        scratch_types=(
            [pltpu.VMEM((nblk, BLK), jnp.int32)] * 2
            + [pltpu.VMEM((n_pad,), jnp.float32)] * nv
            + [pltpu.VMEM((n_pad,), jnp.float32)] * nv
            + [pltpu.VMEM((rpt,), jnp.float32)] * 2
            + [pltpu.VMEM_SHARED((NS, n_pad), jnp.float32)] * nv
        ),
    )
    def seg_kernel(*args):
        vals_hbm = args[:nv]
        src_hbm, dst_hbm = args[nv], args[nv + 1]
        outs = args[nv + 2:2 * nv + 2]
        rest = args[2 * nv + 2:]
        sidx, didx = rest[0], rest[1]
        tabs = rest[2:2 + nv]
        accs = rest[2 + nv:2 + 2 * nv]
        tmp_a, tmp_b = rest[2 + 2 * nv], rest[3 + 2 * nv]
        parts = rest[4 + 2 * nv:]
        c = lax.axis_index("c")
        s = lax.axis_index("s")
        w = c * NS + s
        for v in range(nv):
            pltpu.sync_copy(vals_hbm[v], tabs[v])
            _fill_zeros_1d(accs[v], n_pad)
        pltpu.sync_copy(src_hbm.at[pl.ds(w * nblk, nblk)], sidx)
        pltpu.sync_copy(dst_hbm.at[pl.ds(w * nblk, nblk)], didx)

        def body(k, _):
            b = k // (BLK // LANES)
            j = (k % (BLK // LANES)) * LANES
            si = sidx[b, pl.ds(j, LANES)]
            di = didx[b, pl.ds(j, LANES)]
            for v in range(nv):
                vals = plsc.load_gather(tabs[v], [si])
                plsc.addupdate_scatter(accs[v], [di], vals)
            return 0
        lax.fori_loop(0, ept // LANES, body, 0)
        _reduce_parts_and_emit(c, s, rpt, [
            (accs[v], parts[v], outs[v], tmp_a, tmp_b) for v in range(nv)])

    return seg_kernel


def _make_segsum_kernel(n_pad, e_pad, d):
    ept = e_pad // NW
    nblk = ept // BLK
    rpt = n_pad // NS
    zrows = 8
    mesh = plsc.VectorSubcoreMesh(core_axis_name="c", subcore_axis_name="s")

    @functools.partial(
        pl.kernel,
        out_type=jax.ShapeDtypeStruct((NC, n_pad, d), jnp.float32),
        mesh=mesh,
        scratch_types=[
            pltpu.VMEM((SB, BLK), jnp.int32),
            pltpu.VMEM((SB, BLK), jnp.int32),
            pltpu.VMEM((RING * BLK, d), jnp.float32),
            pltpu.VMEM((zrows, d), jnp.float32),
            pltpu.VMEM_SHARED((n_pad, d), jnp.float32),
        ] + [pltpu.SemaphoreType.DMA] * (2 * RING),
    )
    def seg_kernel(vals_hbm, src_hbm, dst_hbm, out, sidx, didx, rows, zbuf,
                   acc, *sems):
        gsems, ssems = sems[:RING], sems[RING:]
        c = lax.axis_index("c")
        s = lax.axis_index("s")
        w = c * NS + s
        _fill_zeros_2d(zbuf, zrows, d)

        def zero_body(k, _):
            pltpu.sync_copy(zbuf, acc.at[pl.ds(s * rpt + k * zrows, zrows)])
            return 0
        lax.fori_loop(0, rpt // zrows, zero_body, 0)
        plsc.subcore_barrier()

        def body(sb, _):
            base = w * nblk + sb * SB
            pltpu.sync_copy(src_hbm.at[pl.ds(base, SB)], sidx)
            pltpu.sync_copy(dst_hbm.at[pl.ds(base, SB)], didx)
            sdescs = [None] * RING
            gdescs = [None] * RING
            for p in range(SB // RING):
                for r in range(RING):
                    if sdescs[r] is not None:
                        sdescs[r].wait()
                    gdescs[r] = pltpu.async_copy(
                        vals_hbm.at[sidx.at[p * RING + r]],
                        rows.at[pl.ds(r * BLK, BLK)], gsems[r])
                for r in range(RING):
                    gdescs[r].wait()
                    sdescs[r] = pltpu.async_copy(
                        rows.at[pl.ds(r * BLK, BLK)],
                        acc.at[didx.at[p * RING + r]], ssems[r], add=True)
            for r in range(RING):
                sdescs[r].wait()
            return 0
        lax.fori_loop(0, nblk // SB, body, 0)
        plsc.subcore_barrier()
        pltpu.sync_copy(acc.at[pl.ds(s * rpt, rpt)],
                        out.at[c, pl.ds(s * rpt, rpt)])

    return seg_kernel


def _prep_kernel(po0, po1, pi0, pi1, nw, ns_o, nd_o, qa_o, qb_o):
    deg_o = po0[...] + po1[...] + 1.0
    deg_i = pi0[...] + pi1[...] + 1.0
    ns = lax.rsqrt(deg_o)
    nd = lax.rsqrt(deg_i)
    ns_o[...] = ns
    nd_o[...] = nd
    qa_o[...] = nw[...] * ns
    qb_o[...] = ns


def _wprep_kernel(w_lin, b_lin, W0, Wout, w_pred, bout, b_pred,
                  a0_o, c0_o, u_o, k0_o):
    a0_o[...] = jnp.dot(w_lin[...], W0[...], preferred_element_type=jnp.float32)
    c0_o[...] = jnp.dot(b_lin[...], W0[...], preferred_element_type=jnp.float32)
    u_o[...] = jnp.dot(Wout[...], w_pred[...], preferred_element_type=jnp.float32)
    k0_o[...] = jnp.dot(bout[...], w_pred[...],
                        preferred_element_type=jnp.float32) + b_pred[...]


def _h0_kernel(s0, s1, t0, t1, qa, qb, nd, ns, a0, c0, b0, h0_o, y_o):
    s_full = (s0[...] + s1[...] + qa[...]) * nd[...]
    t_full = (t0[...] + t1[...] + qb[...]) * nd[...]
    h0 = jnp.maximum(s_full * a0[...] + t_full * c0[...] + b0[...], 0.0)
    h0_o[...] = h0
    y_o[...] = h0 * ns[...]


def _conv1_kernel(ga, gb, y, W1, b1, nd, ns, h0, u, z_o):
    g = ga[...] + gb[...] + y[...]
    h1 = jnp.maximum(
        jnp.dot(g, W1[...], preferred_element_type=jnp.float32) * nd[...]
        + b1[...], 0.0)
    h = h1 + h0[...]
    z_o[...] = jnp.dot(h * ns[...], u[...], preferred_element_type=jnp.float32)


def _out_kernel(pp0, pp1, z, nd, k0, out_o):
    out_o[...] = (pp0[...] + pp1[...] + z[...]) * nd[...] + k0[...]


def kernel(node_weight, edge_index, w_lin, b_lin, W0, b0, W1, b1, Wout, bout,
           w_pred, b_pred):
    n = node_weight.shape[0]
    e = edge_index.shape[1]
    d = W0.shape[0]
    n_pad = ((n + 1 + 255) // 256) * 256
    egran = NW * BLK * 8
    e_pad = ((e + egran - 1) // egran) * egran

    ei = edge_index.astype(jnp.int32)
    pad = jnp.full((e_pad - e,), n, jnp.int32)
    src = jnp.concatenate([ei[0], pad]).reshape(e_pad // BLK, BLK)
    dst = jnp.concatenate([ei[1], pad]).reshape(e_pad // BLK, BLK)
    nw_pad = jnp.pad(node_weight, (0, n_pad - n))[:, None]

    deg_k = _make_deg_kernel(n_pad, e_pad)
    po, pi = deg_k(src, dst)

    col = jax.ShapeDtypeStruct((n_pad, 1), jnp.float32)
    ns, nd, qa, qb = pl.pallas_call(
        _prep_kernel,
        out_shape=[col, col, col, col],
    )(po[0][:, None], po[1][:, None], pi[0][:, None], pi[1][:, None], nw_pad)

    a0, c0, u, k0 = pl.pallas_call(
        _wprep_kernel,
        out_shape=[
            jax.ShapeDtypeStruct((1, d), jnp.float32),
            jax.ShapeDtypeStruct((1, d), jnp.float32),
            jax.ShapeDtypeStruct((d, 1), jnp.float32),
            jax.ShapeDtypeStruct((1, 1), jnp.float32),
        ],
    )(w_lin[None, :], b_lin[None, :], W0, Wout, w_pred[:, None],
      bout[None, :], b_pred[None, :])

    seg2 = _make_scalar_segsum_kernel(n_pad, e_pad, 2)
    ss, tt = seg2(qa[:, 0], qb[:, 0], src, dst)

    nblocks = n_pad // 128
    colspec = pl.BlockSpec((128, 1), lambda i: (i, 0))
    rowspec = pl.BlockSpec((1, d), lambda i: (0, 0))
    matspec = pl.BlockSpec((128, d), lambda i: (i, 0))
    h0, y = pl.pallas_call(
        _h0_kernel,
        grid=(nblocks,),
        in_specs=[colspec] * 8 + [rowspec] * 3,
        out_specs=[matspec, matspec],
        out_shape=[jax.ShapeDtypeStruct((n_pad, d), jnp.float32)] * 2,
    )(ss[0][:, None], ss[1][:, None], tt[0][:, None], tt[1][:, None],
      qa, qb, nd, ns, a0, c0, b0[None, :])

    seg128 = _make_segsum_kernel(n_pad, e_pad, d)
    g = seg128(y, src, dst)

    wspec = pl.BlockSpec((d, d), lambda i: (0, 0))
    uspec = pl.BlockSpec((d, 1), lambda i: (0, 0))
    z = pl.pallas_call(
        _conv1_kernel,
        grid=(nblocks,),
        in_specs=[matspec, matspec, matspec, wspec, rowspec, colspec, colspec,
                  matspec, uspec],
        out_specs=colspec,
        out_shape=col,
    )(g[0], g[1], y, W1, b1[None, :], nd, ns, h0, u)

    seg1 = _make_scalar_segsum_kernel(n_pad, e_pad, 1)
    (pp,) = seg1(z[:, 0], src, dst)

    k0spec = pl.BlockSpec((1, 1), lambda i: (0, 0))
    logits = pl.pallas_call(
        _out_kernel,
        grid=(nblocks,),
        in_specs=[colspec, colspec, colspec, colspec, k0spec],
        out_specs=colspec,
        out_shape=col,
    )(pp[0][:, None], pp[1][:, None], z, nd, k0)

    return logits[:n, 0]


# speedup vs baseline: 1.5420x; 1.5420x over previous
import functools

import jax
import jax.numpy as jnp
from jax import lax
from jax.experimental import pallas as pl
from jax.experimental.pallas import tpu as pltpu
from jax.experimental.pallas import tpu_sc as plsc

NC = 2
NS = 16
NW = NC * NS
BLK = 128
LANES = 16
RING = 2
SB = 8


def _fill_zeros_1d(ref, n):
    def body(i, _):
        ref[pl.ds(i * LANES, LANES)] = jnp.zeros((LANES,), ref.dtype)
        return 0
    lax.fori_loop(0, n // LANES, body, 0)


def _fill_zeros_2d(ref, rows, cols):
    def body(i, _):
        def inner(j, _):
            ref[i, pl.ds(j * LANES, LANES)] = jnp.zeros((LANES,), ref.dtype)
            return 0
        lax.fori_loop(0, cols // LANES, inner, 0)
        return 0
    lax.fori_loop(0, rows, body, 0)


def _reduce_parts_and_emit(c, s, rpt, triples):
    for acc, parts, _out, _ta, _tb in triples:
        pltpu.sync_copy(acc, parts.at[s])
    plsc.subcore_barrier()
    base = s * rpt
    for _acc, parts, out, ta, tb in triples:
        pltpu.sync_copy(parts.at[0, pl.ds(base, rpt)], ta)

        def tile_body(t, _):
            pltpu.sync_copy(parts.at[t, pl.ds(base, rpt)], tb)

            def add_body(i, _):
                ta[pl.ds(i * LANES, LANES)] = (
                    ta[pl.ds(i * LANES, LANES)] + tb[pl.ds(i * LANES, LANES)])
                return 0
            lax.fori_loop(0, rpt // LANES, add_body, 0)
            return 0
        lax.fori_loop(1, NS, tile_body, 0)
        pltpu.sync_copy(ta, out.at[c, pl.ds(base, rpt)])


def _make_deg_kernel(n_pad, e_pad):
    ept = e_pad // NW
    nblk = ept // BLK
    rpt = n_pad // NS
    mesh = plsc.VectorSubcoreMesh(core_axis_name="c", subcore_axis_name="s")

    @functools.partial(
        pl.kernel,
        out_type=[
            jax.ShapeDtypeStruct((NC, n_pad), jnp.float32),
            jax.ShapeDtypeStruct((NC, n_pad), jnp.float32),
        ],
        mesh=mesh,
        compiler_params=pltpu.CompilerParams(needs_layout_passes=False),
        scratch_types=[
            pltpu.VMEM((nblk, BLK), jnp.int32),
            pltpu.VMEM((nblk, BLK), jnp.int32),
            pltpu.VMEM((n_pad,), jnp.float32),
            pltpu.VMEM((n_pad,), jnp.float32),
            pltpu.VMEM((rpt,), jnp.float32),
            pltpu.VMEM((rpt,), jnp.float32),
            pltpu.VMEM_SHARED((NS, n_pad), jnp.float32),
            pltpu.VMEM_SHARED((NS, n_pad), jnp.float32),
        ],
    )
    def deg_kernel(src_hbm, dst_hbm, out_o, out_i, sidx, didx, acc_o, acc_i,
                   tmp_a, tmp_b, parts_o, parts_i):
        c = lax.axis_index("c")
        s = lax.axis_index("s")
        w = c * NS + s
        _fill_zeros_1d(acc_o, n_pad)
        _fill_zeros_1d(acc_i, n_pad)
        pltpu.sync_copy(src_hbm.at[pl.ds(w * nblk, nblk)], sidx)
        pltpu.sync_copy(dst_hbm.at[pl.ds(w * nblk, nblk)], didx)
        ones = jnp.ones((LANES,), jnp.float32)

        def body(k, _):
            b = k // (BLK // LANES)
            j = (k % (BLK // LANES)) * LANES
            si = sidx[b, pl.ds(j, LANES)]
            di = didx[b, pl.ds(j, LANES)]
            plsc.addupdate_scatter(acc_o, [si], ones)
            plsc.addupdate_scatter(acc_i, [di], ones)
            return 0
        lax.fori_loop(0, ept // LANES, body, 0)
        _reduce_parts_and_emit(c, s, rpt, [
            (acc_o, parts_o, out_o, tmp_a, tmp_b),
            (acc_i, parts_i, out_i, tmp_a, tmp_b),
        ])

    return deg_kernel


def _make_scalar_segsum_kernel(n_pad, e_pad, nv):
    ept = e_pad // NW
    nblk = ept // BLK
    rpt = n_pad // NS
    mesh = plsc.VectorSubcoreMesh(core_axis_name="c", subcore_axis_name="s")

    @functools.partial(
        pl.kernel,
        out_type=[jax.ShapeDtypeStruct((NC, n_pad), jnp.float32)] * nv,
        mesh=mesh,
        compiler_params=pltpu.CompilerParams(needs_layout_passes=False),
        scratch_types=(
            [pltpu.VMEM((nblk, BLK), jnp.int32)] * 2
            + [pltpu.VMEM((n_pad,), jnp.float32)] * nv
            + [pltpu.VMEM((n_pad,), jnp.float32)] * nv
            + [pltpu.VMEM((rpt,), jnp.float32)] * 2
            + [pltpu.VMEM_SHARED((NS, n_pad), jnp.float32)] * nv
        ),
    )
    def seg_kernel(*args):
        vals_hbm = args[:nv]
        src_hbm, dst_hbm = args[nv], args[nv + 1]
        outs = args[nv + 2:2 * nv + 2]
        rest = args[2 * nv + 2:]
        sidx, didx = rest[0], rest[1]
        tabs = rest[2:2 + nv]
        accs = rest[2 + nv:2 + 2 * nv]
        tmp_a, tmp_b = rest[2 + 2 * nv], rest[3 + 2 * nv]
        parts = rest[4 + 2 * nv:]
        c = lax.axis_index("c")
        s = lax.axis_index("s")
        w = c * NS + s
        for v in range(nv):
            pltpu.sync_copy(vals_hbm[v], tabs[v])
            _fill_zeros_1d(accs[v], n_pad)
        pltpu.sync_copy(src_hbm.at[pl.ds(w * nblk, nblk)], sidx)
        pltpu.sync_copy(dst_hbm.at[pl.ds(w * nblk, nblk)], didx)

        def body(k, _):
            b = k // (BLK // LANES)
            j = (k % (BLK // LANES)) * LANES
            si = sidx[b, pl.ds(j, LANES)]
            di = didx[b, pl.ds(j, LANES)]
            for v in range(nv):
                vals = plsc.load_gather(tabs[v], [si])
                plsc.addupdate_scatter(accs[v], [di], vals)
            return 0
        lax.fori_loop(0, ept // LANES, body, 0)
        _reduce_parts_and_emit(c, s, rpt, [
            (accs[v], parts[v], outs[v], tmp_a, tmp_b) for v in range(nv)])

    return seg_kernel


def _make_colsplit_segsum_kernel(n_pad, e_pad, dh):
    nblk = e_pad // NS // BLK
    rpt = n_pad // NS
    zrows = 8
    ring = 4
    mesh = plsc.VectorSubcoreMesh(core_axis_name="c", subcore_axis_name="s")

    @functools.partial(
        pl.kernel,
        out_type=jax.ShapeDtypeStruct((NC, n_pad, dh), jnp.float32),
        mesh=mesh,
        compiler_params=pltpu.CompilerParams(use_tc_tiling_on_sc=False),
        scratch_types=[
            pltpu.VMEM((SB, BLK), jnp.int32),
            pltpu.VMEM((SB, BLK), jnp.int32),
            pltpu.VMEM((ring * BLK, dh), jnp.float32),
            pltpu.VMEM((zrows, dh), jnp.float32),
            pltpu.VMEM_SHARED((n_pad, dh), jnp.float32),
            pltpu.VMEM_SHARED((n_pad, dh), jnp.float32),
        ] + [pltpu.SemaphoreType.DMA] * (2 * ring),
    )
    def seg_kernel(ys_hbm, src_hbm, dst_hbm, out, sidx, didx, rows, zbuf,
                   ytab, acc, *sems):
        gsems, ssems = sems[:ring], sems[ring:]
        c = lax.axis_index("c")
        s = lax.axis_index("s")
        pltpu.sync_copy(ys_hbm.at[c, pl.ds(s * rpt, rpt)],
                        ytab.at[pl.ds(s * rpt, rpt)])
        _fill_zeros_2d(zbuf, zrows, dh)

        def zero_body(k, _):
            pltpu.sync_copy(zbuf, acc.at[pl.ds(s * rpt + k * zrows, zrows)])
            return 0
        lax.fori_loop(0, rpt // zrows, zero_body, 0)
        plsc.subcore_barrier()

        def body(sb, _):
            base = s * nblk + sb * SB
            pltpu.sync_copy(src_hbm.at[pl.ds(base, SB)], sidx)
            pltpu.sync_copy(dst_hbm.at[pl.ds(base, SB)], didx)
            sdescs = [None] * ring
            gdescs = [None] * ring
            for p in range(SB // ring):
                for r in range(ring):
                    if sdescs[r] is not None:
                        sdescs[r].wait()
                    gdescs[r] = pltpu.async_copy(
                        ytab.at[sidx.at[p * ring + r]],
                        rows.at[pl.ds(r * BLK, BLK)], gsems[r])
                for r in range(ring):
                    gdescs[r].wait()
                    sdescs[r] = pltpu.async_copy(
                        rows.at[pl.ds(r * BLK, BLK)],
                        acc.at[didx.at[p * ring + r]], ssems[r], add=True)
            for r in range(ring):
                sdescs[r].wait()
            return 0
        lax.fori_loop(0, nblk // SB, body, 0)
        plsc.subcore_barrier()
        pltpu.sync_copy(acc.at[pl.ds(s * rpt, rpt)],
                        out.at[c, pl.ds(s * rpt, rpt)])

    return seg_kernel


def _make_segsum_kernel(n_pad, e_pad, d):
    ept = e_pad // NW
    nblk = ept // BLK
    rpt = n_pad // NS
    zrows = 8
    mesh = plsc.VectorSubcoreMesh(core_axis_name="c", subcore_axis_name="s")

    @functools.partial(
        pl.kernel,
        out_type=jax.ShapeDtypeStruct((NC, n_pad, d), jnp.float32),
        mesh=mesh,
        scratch_types=[
            pltpu.VMEM((SB, BLK), jnp.int32),
            pltpu.VMEM((SB, BLK), jnp.int32),
            pltpu.VMEM((RING * BLK, d), jnp.float32),
            pltpu.VMEM((zrows, d), jnp.float32),
            pltpu.VMEM_SHARED((n_pad, d), jnp.float32),
        ] + [pltpu.SemaphoreType.DMA] * (2 * RING),
    )
    def seg_kernel(vals_hbm, src_hbm, dst_hbm, out, sidx, didx, rows, zbuf,
                   acc, *sems):
        gsems, ssems = sems[:RING], sems[RING:]
        c = lax.axis_index("c")
        s = lax.axis_index("s")
        w = c * NS + s
        _fill_zeros_2d(zbuf, zrows, d)

        def zero_body(k, _):
            pltpu.sync_copy(zbuf, acc.at[pl.ds(s * rpt + k * zrows, zrows)])
            return 0
        lax.fori_loop(0, rpt // zrows, zero_body, 0)
        plsc.subcore_barrier()

        def body(sb, _):
            base = w * nblk + sb * SB
            pltpu.sync_copy(src_hbm.at[pl.ds(base, SB)], sidx)
            pltpu.sync_copy(dst_hbm.at[pl.ds(base, SB)], didx)
            sdescs = [None] * RING
            gdescs = [None] * RING
            for p in range(SB // RING):
                for r in range(RING):
                    if sdescs[r] is not None:
                        sdescs[r].wait()
                    gdescs[r] = pltpu.async_copy(
                        vals_hbm.at[sidx.at[p * RING + r]],
                        rows.at[pl.ds(r * BLK, BLK)], gsems[r])
                for r in range(RING):
                    gdescs[r].wait()
                    sdescs[r] = pltpu.async_copy(
                        rows.at[pl.ds(r * BLK, BLK)],
                        acc.at[didx.at[p * RING + r]], ssems[r], add=True)
            for r in range(RING):
                sdescs[r].wait()
            return 0
        lax.fori_loop(0, nblk // SB, body, 0)
        plsc.subcore_barrier()
        pltpu.sync_copy(acc.at[pl.ds(s * rpt, rpt)],
                        out.at[c, pl.ds(s * rpt, rpt)])

    return seg_kernel


def _prep_kernel(po0, po1, pi0, pi1, nw, ns_o, nd_o, qa_o, qb_o):
    deg_o = po0[...] + po1[...] + 1.0
    deg_i = pi0[...] + pi1[...] + 1.0
    ns = lax.rsqrt(deg_o)
    nd = lax.rsqrt(deg_i)
    ns_o[...] = ns
    nd_o[...] = nd
    qa_o[...] = nw[...] * ns
    qb_o[...] = ns


def _wprep_kernel(w_lin, b_lin, W0, Wout, w_pred, bout, b_pred,
                  a0_o, c0_o, u_o, k0_o):
    a0_o[...] = jnp.dot(w_lin[...], W0[...], preferred_element_type=jnp.float32)
    c0_o[...] = jnp.dot(b_lin[...], W0[...], preferred_element_type=jnp.float32)
    u_o[...] = jnp.dot(Wout[...], w_pred[...], preferred_element_type=jnp.float32)
    k0_o[...] = jnp.dot(bout[...], w_pred[...],
                        preferred_element_type=jnp.float32) + b_pred[...]


def _h0_kernel(s0, s1, t0, t1, qa, qb, nd, ns, a0, c0, b0, h0_o, y0_o, y1_o):
    s_full = (s0[...] + s1[...] + qa[...]) * nd[...]
    t_full = (t0[...] + t1[...] + qb[...]) * nd[...]
    h0 = jnp.maximum(s_full * a0[...] + t_full * c0[...] + b0[...], 0.0)
    h0_o[...] = h0
    y = h0 * ns[...]
    half = y.shape[1] // 2
    y0_o[...] = y[:, :half]
    y1_o[...] = y[:, half:]


def _conv1_kernel(g0, g1, y0, y1, W1, b1, nd, ns, h0, u, z_o):
    g = jnp.concatenate([g0[...] + y0[...], g1[...] + y1[...]], axis=1)
    h1 = jnp.maximum(
        jnp.dot(g, W1[...], preferred_element_type=jnp.float32) * nd[...]
        + b1[...], 0.0)
    h = h1 + h0[...]
    z_o[...] = jnp.dot(h * ns[...], u[...], preferred_element_type=jnp.float32)


def _out_kernel(pp0, pp1, z, nd, k0, out_o):
    out_o[...] = (pp0[...] + pp1[...] + z[...]) * nd[...] + k0[...]


def kernel(node_weight, edge_index, w_lin, b_lin, W0, b0, W1, b1, Wout, bout,
           w_pred, b_pred):
    n = node_weight.shape[0]
    e = edge_index.shape[1]
    d = W0.shape[0]
    n_pad = ((n + 1 + 255) // 256) * 256
    egran = NW * BLK * 8
    e_pad = ((e + egran - 1) // egran) * egran

    ei = edge_index.astype(jnp.int32)
    pad = jnp.full((e_pad - e,), n, jnp.int32)
    src = jnp.concatenate([ei[0], pad]).reshape(e_pad // BLK, BLK)
    dst = jnp.concatenate([ei[1], pad]).reshape(e_pad // BLK, BLK)
    nw_pad = jnp.pad(node_weight, (0, n_pad - n))[:, None]

    deg_k = _make_deg_kernel(n_pad, e_pad)
    po, pi = deg_k(src, dst)

    col = jax.ShapeDtypeStruct((n_pad, 1), jnp.float32)
    ns, nd, qa, qb = pl.pallas_call(
        _prep_kernel,
        out_shape=[col, col, col, col],
    )(po[0][:, None], po[1][:, None], pi[0][:, None], pi[1][:, None], nw_pad)

    a0, c0, u, k0 = pl.pallas_call(
        _wprep_kernel,
        out_shape=[
            jax.ShapeDtypeStruct((1, d), jnp.float32),
            jax.ShapeDtypeStruct((1, d), jnp.float32),
            jax.ShapeDtypeStruct((d, 1), jnp.float32),
            jax.ShapeDtypeStruct((1, 1), jnp.float32),
        ],
    )(w_lin[None, :], b_lin[None, :], W0, Wout, w_pred[:, None],
      bout[None, :], b_pred[None, :])

    seg2 = _make_scalar_segsum_kernel(n_pad, e_pad, 2)
    ss, tt = seg2(qa[:, 0], qb[:, 0], src, dst)

    nblocks = n_pad // 128
    dh = d // 2
    colspec = pl.BlockSpec((128, 1), lambda i: (i, 0))
    rowspec = pl.BlockSpec((1, d), lambda i: (0, 0))
    matspec = pl.BlockSpec((128, d), lambda i: (i, 0))
    halfspec = pl.BlockSpec((128, dh), lambda i: (i, 0))
    h0, y0, y1 = pl.pallas_call(
        _h0_kernel,
        grid=(nblocks,),
        in_specs=[colspec] * 8 + [rowspec] * 3,
        out_specs=[matspec, halfspec, halfspec],
        out_shape=[jax.ShapeDtypeStruct((n_pad, d), jnp.float32),
                   jax.ShapeDtypeStruct((n_pad, dh), jnp.float32),
                   jax.ShapeDtypeStruct((n_pad, dh), jnp.float32)],
    )(ss[0][:, None], ss[1][:, None], tt[0][:, None], tt[1][:, None],
      qa, qb, nd, ns, a0, c0, b0[None, :])

    seg128 = _make_colsplit_segsum_kernel(n_pad, e_pad, dh)
    ystack = jnp.stack([y0, y1])
    g = seg128(ystack, src, dst)

    wspec = pl.BlockSpec((d, d), lambda i: (0, 0))
    uspec = pl.BlockSpec((d, 1), lambda i: (0, 0))
    z = pl.pallas_call(
        _conv1_kernel,
        grid=(nblocks,),
        in_specs=[halfspec, halfspec, halfspec, halfspec, wspec, rowspec,
                  colspec, colspec, matspec, uspec],
        out_specs=colspec,
        out_shape=col,
    )(g[0], g[1], y0, y1, W1, b1[None, :], nd, ns, h0, u)

    seg1 = _make_scalar_segsum_kernel(n_pad, e_pad, 1)
    (pp,) = seg1(z[:, 0], src, dst)

    k0spec = pl.BlockSpec((1, 1), lambda i: (0, 0))
    logits = pl.pallas_call(
        _out_kernel,
        grid=(nblocks,),
        in_specs=[colspec, colspec, colspec, colspec, k0spec],
        out_specs=colspec,
        out_shape=col,
    )(pp[0][:, None], pp[1][:, None], z, nd, k0)

    return logits[:n, 0]


# speedup vs baseline: 1.9721x; 1.2789x over previous
import functools

import jax
import jax.numpy as jnp
from jax import lax
from jax.experimental import pallas as pl
from jax.experimental.pallas import tpu as pltpu
from jax.experimental.pallas import tpu_sc as plsc

NC = 2
NS = 16
NW = NC * NS
BLK = 128
LANES = 16
RING = 2
SB = 8


def _fill_zeros_1d(ref, n):
    def body(i, _):
        ref[pl.ds(i * LANES, LANES)] = jnp.zeros((LANES,), ref.dtype)
        return 0
    lax.fori_loop(0, n // LANES, body, 0)


def _fill_zeros_2d(ref, rows, cols):
    def body(i, _):
        def inner(j, _):
            ref[i, pl.ds(j * LANES, LANES)] = jnp.zeros((LANES,), ref.dtype)
            return 0
        lax.fori_loop(0, cols // LANES, inner, 0)
        return 0
    lax.fori_loop(0, rows, body, 0)


def _reduce_parts_and_emit(c, s, rpt, triples):
    for acc, parts, _out, _ta, _tb in triples:
        pltpu.sync_copy(acc, parts.at[s])
    plsc.subcore_barrier()
    base = s * rpt
    for _acc, parts, out, ta, tb in triples:
        pltpu.sync_copy(parts.at[0, pl.ds(base, rpt)], ta)

        def tile_body(t, _):
            pltpu.sync_copy(parts.at[t, pl.ds(base, rpt)], tb)

            def add_body(i, _):
                ta[pl.ds(i * LANES, LANES)] = (
                    ta[pl.ds(i * LANES, LANES)] + tb[pl.ds(i * LANES, LANES)])
                return 0
            lax.fori_loop(0, rpt // LANES, add_body, 0)
            return 0
        lax.fori_loop(1, NS, tile_body, 0)
        pltpu.sync_copy(ta, out.at[c, pl.ds(base, rpt)])


def _make_deg_kernel(n_pad, e_pad):
    ept = e_pad // NW
    nblk = ept // BLK
    rpt = n_pad // NS
    mesh = plsc.VectorSubcoreMesh(core_axis_name="c", subcore_axis_name="s")

    @functools.partial(
        pl.kernel,
        out_type=[
            jax.ShapeDtypeStruct((NC, n_pad), jnp.float32),
            jax.ShapeDtypeStruct((NC, n_pad), jnp.float32),
        ],
        mesh=mesh,
        compiler_params=pltpu.CompilerParams(needs_layout_passes=False),
        scratch_types=[
            pltpu.VMEM((nblk, BLK), jnp.int32),
            pltpu.VMEM((nblk, BLK), jnp.int32),
            pltpu.VMEM((n_pad,), jnp.float32),
            pltpu.VMEM((n_pad,), jnp.float32),
            pltpu.VMEM((rpt,), jnp.float32),
            pltpu.VMEM((rpt,), jnp.float32),
            pltpu.VMEM_SHARED((NS, n_pad), jnp.float32),
            pltpu.VMEM_SHARED((NS, n_pad), jnp.float32),
        ],
    )
    def deg_kernel(src_hbm, dst_hbm, out_o, out_i, sidx, didx, acc_o, acc_i,
                   tmp_a, tmp_b, parts_o, parts_i):
        c = lax.axis_index("c")
        s = lax.axis_index("s")
        w = c * NS + s
        _fill_zeros_1d(acc_o, n_pad)
        _fill_zeros_1d(acc_i, n_pad)
        pltpu.sync_copy(src_hbm.at[pl.ds(w * nblk, nblk)], sidx)
        pltpu.sync_copy(dst_hbm.at[pl.ds(w * nblk, nblk)], didx)
        ones = jnp.ones((LANES,), jnp.float32)

        def body(k, _):
            b = k // (BLK // LANES)
            j = (k % (BLK // LANES)) * LANES
            si = sidx[b, pl.ds(j, LANES)]
            di = didx[b, pl.ds(j, LANES)]
            plsc.addupdate_scatter(acc_o, [si], ones)
            plsc.addupdate_scatter(acc_i, [di], ones)
            return 0
        lax.fori_loop(0, ept // LANES, body, 0)
        _reduce_parts_and_emit(c, s, rpt, [
            (acc_o, parts_o, out_o, tmp_a, tmp_b),
            (acc_i, parts_i, out_i, tmp_a, tmp_b),
        ])

    return deg_kernel


def _make_scalar_segsum_kernel(n_pad, e_pad, nv):
    ept = e_pad // NW
    nblk = ept // BLK
    rpt = n_pad // NS
    mesh = plsc.VectorSubcoreMesh(core_axis_name="c", subcore_axis_name="s")

    @functools.partial(
        pl.kernel,
        out_type=[jax.ShapeDtypeStruct((NC, n_pad), jnp.float32)] * nv,
        mesh=mesh,
        compiler_params=pltpu.CompilerParams(needs_layout_passes=False),
        scratch_types=(
            [pltpu.VMEM((nblk, BLK), jnp.int32)] * 2
            + [pltpu.VMEM((n_pad,), jnp.float32)] * nv
            + [pltpu.VMEM((n_pad,), jnp.float32)] * nv
            + [pltpu.VMEM((rpt,), jnp.float32)] * 2
            + [pltpu.VMEM_SHARED((NS, n_pad), jnp.float32)] * nv
        ),
    )
    def seg_kernel(*args):
        vals_hbm = args[:nv]
        src_hbm, dst_hbm = args[nv], args[nv + 1]
        outs = args[nv + 2:2 * nv + 2]
        rest = args[2 * nv + 2:]
        sidx, didx = rest[0], rest[1]
        tabs = rest[2:2 + nv]
        accs = rest[2 + nv:2 + 2 * nv]
        tmp_a, tmp_b = rest[2 + 2 * nv], rest[3 + 2 * nv]
        parts = rest[4 + 2 * nv:]
        c = lax.axis_index("c")
        s = lax.axis_index("s")
        w = c * NS + s
        for v in range(nv):
            pltpu.sync_copy(vals_hbm[v], tabs[v])
            _fill_zeros_1d(accs[v], n_pad)
        pltpu.sync_copy(src_hbm.at[pl.ds(w * nblk, nblk)], sidx)
        pltpu.sync_copy(dst_hbm.at[pl.ds(w * nblk, nblk)], didx)

        def body(k, _):
            b = k // (BLK // LANES)
            j = (k % (BLK // LANES)) * LANES
            si = sidx[b, pl.ds(j, LANES)]
            di = didx[b, pl.ds(j, LANES)]
            for v in range(nv):
                vals = plsc.load_gather(tabs[v], [si])
                plsc.addupdate_scatter(accs[v], [di], vals)
            return 0
        lax.fori_loop(0, ept // LANES, body, 0)
        _reduce_parts_and_emit(c, s, rpt, [
            (accs[v], parts[v], outs[v], tmp_a, tmp_b) for v in range(nv)])

    return seg_kernel


def _make_colsplit_segsum_kernel(n_pad, e_pad, dh):
    nblk = e_pad // NS // BLK
    rpt = n_pad // NS
    zrows = 8
    ring = 4
    mesh = plsc.VectorSubcoreMesh(core_axis_name="c", subcore_axis_name="s")

    @functools.partial(
        pl.kernel,
        out_type=jax.ShapeDtypeStruct((NC, n_pad, dh), jnp.float32),
        mesh=mesh,
        compiler_params=pltpu.CompilerParams(use_tc_tiling_on_sc=False),
        scratch_types=[
            pltpu.VMEM((SB, BLK), jnp.int32),
            pltpu.VMEM((SB, BLK), jnp.int32),
            pltpu.VMEM((ring * BLK, dh), jnp.float32),
            pltpu.VMEM((zrows, dh), jnp.float32),
            pltpu.VMEM_SHARED((n_pad, dh), jnp.float32),
            pltpu.VMEM_SHARED((n_pad, dh), jnp.float32),
        ] + [pltpu.SemaphoreType.DMA] * (2 * ring),
    )
    def seg_kernel(ys_hbm, src_hbm, dst_hbm, out, sidx, didx, rows, zbuf,
                   ytab, acc, *sems):
        gsems, ssems = sems[:ring], sems[ring:]
        c = lax.axis_index("c")
        s = lax.axis_index("s")
        pltpu.sync_copy(ys_hbm.at[c, pl.ds(s * rpt, rpt)],
                        ytab.at[pl.ds(s * rpt, rpt)])
        _fill_zeros_2d(zbuf, zrows, dh)

        def zero_body(k, _):
            pltpu.sync_copy(zbuf, acc.at[pl.ds(s * rpt + k * zrows, zrows)])
            return 0
        lax.fori_loop(0, rpt // zrows, zero_body, 0)
        plsc.subcore_barrier()

        def body(sb, _):
            base = s * nblk + sb * SB
            pltpu.sync_copy(src_hbm.at[pl.ds(base, SB)], sidx)
            pltpu.sync_copy(dst_hbm.at[pl.ds(base, SB)], didx)
            sdescs = [None] * ring
            gdescs = [None] * ring
            for p in range(SB // ring):
                for r in range(ring):
                    if sdescs[r] is not None:
                        sdescs[r].wait()
                    gdescs[r] = pltpu.async_copy(
                        ytab.at[sidx.at[p * ring + r]],
                        rows.at[pl.ds(r * BLK, BLK)], gsems[r])
                for r in range(ring):
                    gdescs[r].wait()
                    sdescs[r] = pltpu.async_copy(
                        rows.at[pl.ds(r * BLK, BLK)],
                        acc.at[didx.at[p * ring + r]], ssems[r], add=True)
            for r in range(ring):
                sdescs[r].wait()
            return 0
        lax.fori_loop(0, nblk // SB, body, 0)
        plsc.subcore_barrier()
        pltpu.sync_copy(acc.at[pl.ds(s * rpt, rpt)],
                        out.at[c, pl.ds(s * rpt, rpt)])

    return seg_kernel


def _make_segsum_kernel(n_pad, e_pad, d):
    ept = e_pad // NW
    nblk = ept // BLK
    rpt = n_pad // NS
    zrows = 8
    mesh = plsc.VectorSubcoreMesh(core_axis_name="c", subcore_axis_name="s")

    @functools.partial(
        pl.kernel,
        out_type=jax.ShapeDtypeStruct((NC, n_pad, d), jnp.float32),
        mesh=mesh,
        scratch_types=[
            pltpu.VMEM((SB, BLK), jnp.int32),
            pltpu.VMEM((SB, BLK), jnp.int32),
            pltpu.VMEM((RING * BLK, d), jnp.float32),
            pltpu.VMEM((zrows, d), jnp.float32),
            pltpu.VMEM_SHARED((n_pad, d), jnp.float32),
        ] + [pltpu.SemaphoreType.DMA] * (2 * RING),
    )
    def seg_kernel(vals_hbm, src_hbm, dst_hbm, out, sidx, didx, rows, zbuf,
                   acc, *sems):
        gsems, ssems = sems[:RING], sems[RING:]
        c = lax.axis_index("c")
        s = lax.axis_index("s")
        w = c * NS + s
        _fill_zeros_2d(zbuf, zrows, d)

        def zero_body(k, _):
            pltpu.sync_copy(zbuf, acc.at[pl.ds(s * rpt + k * zrows, zrows)])
            return 0
        lax.fori_loop(0, rpt // zrows, zero_body, 0)
        plsc.subcore_barrier()

        def body(sb, _):
            base = w * nblk + sb * SB
            pltpu.sync_copy(src_hbm.at[pl.ds(base, SB)], sidx)
            pltpu.sync_copy(dst_hbm.at[pl.ds(base, SB)], didx)
            sdescs = [None] * RING
            gdescs = [None] * RING
            for p in range(SB // RING):
                for r in range(RING):
                    if sdescs[r] is not None:
                        sdescs[r].wait()
                    gdescs[r] = pltpu.async_copy(
                        vals_hbm.at[sidx.at[p * RING + r]],
                        rows.at[pl.ds(r * BLK, BLK)], gsems[r])
                for r in range(RING):
                    gdescs[r].wait()
                    sdescs[r] = pltpu.async_copy(
                        rows.at[pl.ds(r * BLK, BLK)],
                        acc.at[didx.at[p * RING + r]], ssems[r], add=True)
            for r in range(RING):
                sdescs[r].wait()
            return 0
        lax.fori_loop(0, nblk // SB, body, 0)
        plsc.subcore_barrier()
        pltpu.sync_copy(acc.at[pl.ds(s * rpt, rpt)],
                        out.at[c, pl.ds(s * rpt, rpt)])

    return seg_kernel


def _prep_kernel(po0, po1, pi0, pi1, nw, ns_o, nd_o, qa_o, qb_o):
    deg_o = po0[...] + po1[...] + 1.0
    deg_i = pi0[...] + pi1[...] + 1.0
    ns = lax.rsqrt(deg_o)
    nd = lax.rsqrt(deg_i)
    ns_o[...] = ns
    nd_o[...] = nd
    qa_o[...] = nw[...] * ns
    qb_o[...] = ns


def _wprep_kernel(w_lin, b_lin, W0, Wout, w_pred, bout, b_pred,
                  a0_o, c0_o, u_o, k0_o):
    a0_o[...] = jnp.dot(w_lin[...], W0[...], preferred_element_type=jnp.float32)
    c0_o[...] = jnp.dot(b_lin[...], W0[...], preferred_element_type=jnp.float32)
    u_o[...] = jnp.dot(Wout[...], w_pred[...], preferred_element_type=jnp.float32)
    k0_o[...] = jnp.dot(bout[...], w_pred[...],
                        preferred_element_type=jnp.float32) + b_pred[...]


def _h0_kernel(s0, s1, t0, t1, qa, qb, nd, ns, a0, c0, b0, h0_o, ys_o):
    s_full = (s0[...] + s1[...] + qa[...]) * nd[...]
    t_full = (t0[...] + t1[...] + qb[...]) * nd[...]
    h0 = jnp.maximum(s_full * a0[...] + t_full * c0[...] + b0[...], 0.0)
    h0_o[...] = h0
    y = h0 * ns[...]
    half = y.shape[1] // 2
    ys_o[0] = y[:, :half]
    ys_o[1] = y[:, half:]


def _conv1_kernel(g0, g1, y0, y1, W1, b1, nd, ns, h0, u, z_o):
    g = jnp.concatenate([g0[0] + y0[0], g1[0] + y1[0]], axis=1)
    h1 = jnp.maximum(
        jnp.dot(g, W1[...], preferred_element_type=jnp.float32) * nd[...]
        + b1[...], 0.0)
    h = h1 + h0[...]
    z_o[...] = jnp.dot(h * ns[...], u[...], preferred_element_type=jnp.float32)


def _out_kernel(pp0, pp1, z, nd, k0, out_o):
    out_o[...] = (pp0[...] + pp1[...] + z[...]) * nd[...] + k0[...]


def kernel(node_weight, edge_index, w_lin, b_lin, W0, b0, W1, b1, Wout, bout,
           w_pred, b_pred):
    n = node_weight.shape[0]
    e = edge_index.shape[1]
    d = W0.shape[0]
    n_pad = ((n + 1 + 255) // 256) * 256
    egran = NW * BLK * 8
    e_pad = ((e + egran - 1) // egran) * egran

    ei = edge_index.astype(jnp.int32)
    pad = jnp.full((e_pad - e,), n, jnp.int32)
    src = jnp.concatenate([ei[0], pad]).reshape(e_pad // BLK, BLK)
    dst = jnp.concatenate([ei[1], pad]).reshape(e_pad // BLK, BLK)
    nw_pad = jnp.pad(node_weight, (0, n_pad - n))[:, None]

    deg_k = _make_deg_kernel(n_pad, e_pad)
    po, pi = deg_k(src, dst)

    col = jax.ShapeDtypeStruct((n_pad, 1), jnp.float32)
    ns, nd, qa, qb = pl.pallas_call(
        _prep_kernel,
        out_shape=[col, col, col, col],
    )(po[0][:, None], po[1][:, None], pi[0][:, None], pi[1][:, None], nw_pad)

    a0, c0, u, k0 = pl.pallas_call(
        _wprep_kernel,
        out_shape=[
            jax.ShapeDtypeStruct((1, d), jnp.float32),
            jax.ShapeDtypeStruct((1, d), jnp.float32),
            jax.ShapeDtypeStruct((d, 1), jnp.float32),
            jax.ShapeDtypeStruct((1, 1), jnp.float32),
        ],
    )(w_lin[None, :], b_lin[None, :], W0, Wout, w_pred[:, None],
      bout[None, :], b_pred[None, :])

    seg2 = _make_scalar_segsum_kernel(n_pad, e_pad, 2)
    ss, tt = seg2(qa[:, 0], qb[:, 0], src, dst)

    rblk = 1024
    nblocks = n_pad // rblk
    dh = d // 2
    colspec = pl.BlockSpec((rblk, 1), lambda i: (i, 0))
    rowspec = pl.BlockSpec((1, d), lambda i: (0, 0))
    matspec = pl.BlockSpec((rblk, d), lambda i: (i, 0))
    ysspec = pl.BlockSpec((NC, rblk, dh), lambda i: (0, i, 0))
    h0, ys = pl.pallas_call(
        _h0_kernel,
        grid=(nblocks,),
        in_specs=[colspec] * 8 + [rowspec] * 3,
        out_specs=[matspec, ysspec],
        out_shape=[jax.ShapeDtypeStruct((n_pad, d), jnp.float32),
                   jax.ShapeDtypeStruct((NC, n_pad, dh), jnp.float32)],
    )(ss[0][:, None], ss[1][:, None], tt[0][:, None], tt[1][:, None],
      qa, qb, nd, ns, a0, c0, b0[None, :])

    seg128 = _make_colsplit_segsum_kernel(n_pad, e_pad, dh)
    g = seg128(ys, src, dst)

    wspec = pl.BlockSpec((d, d), lambda i: (0, 0))
    uspec = pl.BlockSpec((d, 1), lambda i: (0, 0))
    h0spec = pl.BlockSpec((1, rblk, dh), lambda i: (0, i, 0))
    h1spec = pl.BlockSpec((1, rblk, dh), lambda i: (1, i, 0))
    z = pl.pallas_call(
        _conv1_kernel,
        grid=(nblocks,),
        in_specs=[h0spec, h1spec, h0spec, h1spec, wspec, rowspec,
                  colspec, colspec, matspec, uspec],
        out_specs=colspec,
        out_shape=col,
    )(g, g, ys, ys, W1, b1[None, :], nd, ns, h0, u)

    seg1 = _make_scalar_segsum_kernel(n_pad, e_pad, 1)
    (pp,) = seg1(z[:, 0], src, dst)

    logits = pl.pallas_call(
        _out_kernel,
        out_shape=col,
    )(pp[0][:, None], pp[1][:, None], z, nd, k0)

    return logits[:n, 0]


# speedup vs baseline: 2.0583x; 1.0437x over previous
import functools

import jax
import jax.numpy as jnp
from jax import lax
from jax.experimental import pallas as pl
from jax.experimental.pallas import tpu as pltpu
from jax.experimental.pallas import tpu_sc as plsc

NC = 2
NS = 16
NW = NC * NS
BLK = 128
LANES = 16
RING = 2
SB = 8


def _fill_zeros_1d(ref, n):
    def body(i, _):
        ref[pl.ds(i * LANES, LANES)] = jnp.zeros((LANES,), ref.dtype)
        return 0
    lax.fori_loop(0, n // LANES, body, 0)


def _fill_zeros_2d(ref, rows, cols):
    def body(i, _):
        def inner(j, _):
            ref[i, pl.ds(j * LANES, LANES)] = jnp.zeros((LANES,), ref.dtype)
            return 0
        lax.fori_loop(0, cols // LANES, inner, 0)
        return 0
    lax.fori_loop(0, rows, body, 0)


def _reduce_parts_and_emit(c, s, rpt, triples):
    for acc, parts, _out, _tmp, _ta in triples:
        pltpu.sync_copy(acc, parts.at[s])
    plsc.subcore_barrier()
    base = s * rpt
    for _acc, parts, out, tmp, ta in triples:
        pltpu.sync_copy(parts.at[:, pl.ds(base, rpt)], tmp)

        def add_body(i, _):
            acc16 = tmp[0, pl.ds(i * LANES, LANES)]
            for t in range(1, NS):
                acc16 = acc16 + tmp[t, pl.ds(i * LANES, LANES)]
            ta[pl.ds(i * LANES, LANES)] = acc16
            return 0
        lax.fori_loop(0, rpt // LANES, add_body, 0)
        pltpu.sync_copy(ta, out.at[c, pl.ds(base, rpt)])


def _make_deg_kernel(n_pad, e_pad):
    ept = e_pad // NW
    nblk = ept // BLK
    rpt = n_pad // NS
    mesh = plsc.VectorSubcoreMesh(core_axis_name="c", subcore_axis_name="s")

    @functools.partial(
        pl.kernel,
        out_type=[
            jax.ShapeDtypeStruct((NC, n_pad), jnp.float32),
            jax.ShapeDtypeStruct((NC, n_pad), jnp.float32),
        ],
        mesh=mesh,
        compiler_params=pltpu.CompilerParams(needs_layout_passes=False),
        scratch_types=[
            pltpu.VMEM((nblk, BLK), jnp.int32),
            pltpu.VMEM((nblk, BLK), jnp.int32),
            pltpu.VMEM((n_pad,), jnp.float32),
            pltpu.VMEM((n_pad,), jnp.float32),
            pltpu.VMEM((NS, n_pad // NS), jnp.float32),
            pltpu.VMEM((n_pad // NS,), jnp.float32),
            pltpu.VMEM_SHARED((NS, n_pad), jnp.float32),
            pltpu.VMEM_SHARED((NS, n_pad), jnp.float32),
        ],
    )
    def deg_kernel(src_hbm, dst_hbm, out_o, out_i, sidx, didx, acc_o, acc_i,
                   tmp_a, tmp_b, parts_o, parts_i):
        c = lax.axis_index("c")
        s = lax.axis_index("s")
        w = c * NS + s
        _fill_zeros_1d(acc_o, n_pad)
        _fill_zeros_1d(acc_i, n_pad)
        pltpu.sync_copy(src_hbm.at[pl.ds(w * nblk, nblk)], sidx)
        pltpu.sync_copy(dst_hbm.at[pl.ds(w * nblk, nblk)], didx)
        ones = jnp.ones((LANES,), jnp.float32)

        def body(k, _):
            b = k // (BLK // LANES)
            j = (k % (BLK // LANES)) * LANES
            si = sidx[b, pl.ds(j, LANES)]
            di = didx[b, pl.ds(j, LANES)]
            plsc.addupdate_scatter(acc_o, [si], ones)
            plsc.addupdate_scatter(acc_i, [di], ones)
            return 0
        lax.fori_loop(0, ept // LANES, body, 0)
        _reduce_parts_and_emit(c, s, rpt, [
            (acc_o, parts_o, out_o, tmp_a, tmp_b),
            (acc_i, parts_i, out_i, tmp_a, tmp_b),
        ])

    return deg_kernel


def _make_scalar_segsum_kernel(n_pad, e_pad, nv):
    ept = e_pad // NW
    nblk = ept // BLK
    rpt = n_pad // NS
    mesh = plsc.VectorSubcoreMesh(core_axis_name="c", subcore_axis_name="s")

    @functools.partial(
        pl.kernel,
        out_type=[jax.ShapeDtypeStruct((NC, n_pad), jnp.float32)] * nv,
        mesh=mesh,
        compiler_params=pltpu.CompilerParams(needs_layout_passes=False),
        scratch_types=(
            [pltpu.VMEM((nblk, BLK), jnp.int32)] * 2
            + [pltpu.VMEM((n_pad,), jnp.float32)] * nv
            + [pltpu.VMEM((n_pad,), jnp.float32)] * nv
            + [pltpu.VMEM((NS, n_pad // NS), jnp.float32),
               pltpu.VMEM((n_pad // NS,), jnp.float32)]
            + [pltpu.VMEM_SHARED((NS, n_pad), jnp.float32)] * nv
        ),
    )
    def seg_kernel(*args):
        vals_hbm = args[:nv]
        src_hbm, dst_hbm = args[nv], args[nv + 1]
        outs = args[nv + 2:2 * nv + 2]
        rest = args[2 * nv + 2:]
        sidx, didx = rest[0], rest[1]
        tabs = rest[2:2 + nv]
        accs = rest[2 + nv:2 + 2 * nv]
        tmp_a, tmp_b = rest[2 + 2 * nv], rest[3 + 2 * nv]
        parts = rest[4 + 2 * nv:]
        c = lax.axis_index("c")
        s = lax.axis_index("s")
        w = c * NS + s
        for v in range(nv):
            pltpu.sync_copy(vals_hbm[v], tabs[v])
            _fill_zeros_1d(accs[v], n_pad)
        pltpu.sync_copy(src_hbm.at[pl.ds(w * nblk, nblk)], sidx)
        pltpu.sync_copy(dst_hbm.at[pl.ds(w * nblk, nblk)], didx)

        def body(k, _):
            b = k // (BLK // LANES)
            j = (k % (BLK // LANES)) * LANES
            si = sidx[b, pl.ds(j, LANES)]
            di = didx[b, pl.ds(j, LANES)]
            for v in range(nv):
                vals = plsc.load_gather(tabs[v], [si])
                plsc.addupdate_scatter(accs[v], [di], vals)
            return 0
        lax.fori_loop(0, ept // LANES, body, 0)
        _reduce_parts_and_emit(c, s, rpt, [
            (accs[v], parts[v], outs[v], tmp_a, tmp_b) for v in range(nv)])

    return seg_kernel


def _make_colsplit_segsum_kernel(n_pad, e_pad, dh):
    nblk = e_pad // NS // BLK
    rpt = n_pad // NS
    zrows = 8
    ring = 4
    mesh = plsc.VectorSubcoreMesh(core_axis_name="c", subcore_axis_name="s")

    @functools.partial(
        pl.kernel,
        out_type=jax.ShapeDtypeStruct((NC, n_pad, dh), jnp.float32),
        mesh=mesh,
        compiler_params=pltpu.CompilerParams(use_tc_tiling_on_sc=False),
        scratch_types=[
            pltpu.VMEM((SB, BLK), jnp.int32),
            pltpu.VMEM((SB, BLK), jnp.int32),
            pltpu.VMEM((ring * BLK, dh), jnp.float32),
            pltpu.VMEM((zrows, dh), jnp.float32),
            pltpu.VMEM_SHARED((n_pad, dh), jnp.float32),
            pltpu.VMEM_SHARED((n_pad, dh), jnp.float32),
        ] + [pltpu.SemaphoreType.DMA] * (2 * ring),
    )
    def seg_kernel(ys_hbm, src_hbm, dst_hbm, out, sidx, didx, rows, zbuf,
                   ytab, acc, *sems):
        gsems, ssems = sems[:ring], sems[ring:]
        c = lax.axis_index("c")
        s = lax.axis_index("s")
        pltpu.sync_copy(ys_hbm.at[c, pl.ds(s * rpt, rpt)],
                        ytab.at[pl.ds(s * rpt, rpt)])
        _fill_zeros_2d(zbuf, zrows, dh)

        def zero_body(k, _):
            pltpu.sync_copy(zbuf, acc.at[pl.ds(s * rpt + k * zrows, zrows)])
            return 0
        lax.fori_loop(0, rpt // zrows, zero_body, 0)
        plsc.subcore_barrier()

        def body(sb, _):
            base = s * nblk + sb * SB
            pltpu.sync_copy(src_hbm.at[pl.ds(base, SB)], sidx)
            pltpu.sync_copy(dst_hbm.at[pl.ds(base, SB)], didx)
            sdescs = [None] * ring
            gdescs = [None] * ring
            for p in range(SB // ring):
                for r in range(ring):
                    if sdescs[r] is not None:
                        sdescs[r].wait()
                    gdescs[r] = pltpu.async_copy(
                        ytab.at[sidx.at[p * ring + r]],
                        rows.at[pl.ds(r * BLK, BLK)], gsems[r])
                for r in range(ring):
                    gdescs[r].wait()
                    sdescs[r] = pltpu.async_copy(
                        rows.at[pl.ds(r * BLK, BLK)],
                        acc.at[didx.at[p * ring + r]], ssems[r], add=True)
            for r in range(ring):
                sdescs[r].wait()
            return 0
        lax.fori_loop(0, nblk // SB, body, 0)
        plsc.subcore_barrier()
        pltpu.sync_copy(acc.at[pl.ds(s * rpt, rpt)],
                        out.at[c, pl.ds(s * rpt, rpt)])

    return seg_kernel


def _make_segsum_kernel(n_pad, e_pad, d):
    ept = e_pad // NW
    nblk = ept // BLK
    rpt = n_pad // NS
    zrows = 8
    mesh = plsc.VectorSubcoreMesh(core_axis_name="c", subcore_axis_name="s")

    @functools.partial(
        pl.kernel,
        out_type=jax.ShapeDtypeStruct((NC, n_pad, d), jnp.float32),
        mesh=mesh,
        scratch_types=[
            pltpu.VMEM((SB, BLK), jnp.int32),
            pltpu.VMEM((SB, BLK), jnp.int32),
            pltpu.VMEM((RING * BLK, d), jnp.float32),
            pltpu.VMEM((zrows, d), jnp.float32),
            pltpu.VMEM_SHARED((n_pad, d), jnp.float32),
        ] + [pltpu.SemaphoreType.DMA] * (2 * RING),
    )
    def seg_kernel(vals_hbm, src_hbm, dst_hbm, out, sidx, didx, rows, zbuf,
                   acc, *sems):
        gsems, ssems = sems[:RING], sems[RING:]
        c = lax.axis_index("c")
        s = lax.axis_index("s")
        w = c * NS + s
        _fill_zeros_2d(zbuf, zrows, d)

        def zero_body(k, _):
            pltpu.sync_copy(zbuf, acc.at[pl.ds(s * rpt + k * zrows, zrows)])
            return 0
        lax.fori_loop(0, rpt // zrows, zero_body, 0)
        plsc.subcore_barrier()

        def body(sb, _):
            base = w * nblk + sb * SB
            pltpu.sync_copy(src_hbm.at[pl.ds(base, SB)], sidx)
            pltpu.sync_copy(dst_hbm.at[pl.ds(base, SB)], didx)
            sdescs = [None] * RING
            gdescs = [None] * RING
            for p in range(SB // RING):
                for r in range(RING):
                    if sdescs[r] is not None:
                        sdescs[r].wait()
                    gdescs[r] = pltpu.async_copy(
                        vals_hbm.at[sidx.at[p * RING + r]],
                        rows.at[pl.ds(r * BLK, BLK)], gsems[r])
                for r in range(RING):
                    gdescs[r].wait()
                    sdescs[r] = pltpu.async_copy(
                        rows.at[pl.ds(r * BLK, BLK)],
                        acc.at[didx.at[p * RING + r]], ssems[r], add=True)
            for r in range(RING):
                sdescs[r].wait()
            return 0
        lax.fori_loop(0, nblk // SB, body, 0)
        plsc.subcore_barrier()
        pltpu.sync_copy(acc.at[pl.ds(s * rpt, rpt)],
                        out.at[c, pl.ds(s * rpt, rpt)])

    return seg_kernel


def _prep_kernel(po0, po1, pi0, pi1, nw, ns_o, nd_o, qa_o, qb_o):
    deg_o = po0[...] + po1[...] + 1.0
    deg_i = pi0[...] + pi1[...] + 1.0
    ns = lax.rsqrt(deg_o)
    nd = lax.rsqrt(deg_i)
    ns_o[...] = ns
    nd_o[...] = nd
    qa_o[...] = nw[...] * ns
    qb_o[...] = ns


def _wprep_kernel(w_lin, b_lin, W0, Wout, w_pred, bout, b_pred,
                  a0_o, c0_o, u_o, k0_o):
    a0_o[...] = jnp.dot(w_lin[...], W0[...], preferred_element_type=jnp.float32)
    c0_o[...] = jnp.dot(b_lin[...], W0[...], preferred_element_type=jnp.float32)
    u_o[...] = jnp.dot(Wout[...], w_pred[...], preferred_element_type=jnp.float32)
    k0_o[...] = jnp.dot(bout[...], w_pred[...],
                        preferred_element_type=jnp.float32) + b_pred[...]


def _h0_kernel(s0, s1, t0, t1, qa, qb, nd, ns, a0, c0, b0, h0_o, ys_o):
    s_full = (s0[...] + s1[...] + qa[...]) * nd[...]
    t_full = (t0[...] + t1[...] + qb[...]) * nd[...]
    h0 = jnp.maximum(s_full * a0[...] + t_full * c0[...] + b0[...], 0.0)
    h0_o[...] = h0
    y = h0 * ns[...]
    half = y.shape[1] // 2
    ys_o[0] = y[:, :half]
    ys_o[1] = y[:, half:]


def _conv1_kernel(g0, g1, y0, y1, W1, b1, nd, ns, h0, u, z_o):
    g = jnp.concatenate([g0[0] + y0[0], g1[0] + y1[0]], axis=1)
    h1 = jnp.maximum(
        jnp.dot(g, W1[...], preferred_element_type=jnp.float32) * nd[...]
        + b1[...], 0.0)
    h = h1 + h0[...]
    z_o[...] = jnp.dot(h * ns[...], u[...], preferred_element_type=jnp.float32)


def _out_kernel(pp0, pp1, z, nd, k0, out_o):
    out_o[...] = (pp0[...] + pp1[...] + z[...]) * nd[...] + k0[...]


def kernel(node_weight, edge_index, w_lin, b_lin, W0, b0, W1, b1, Wout, bout,
           w_pred, b_pred):
    n = node_weight.shape[0]
    e = edge_index.shape[1]
    d = W0.shape[0]
    n_pad = ((n + 1 + 255) // 256) * 256
    egran = NW * BLK * 8
    e_pad = ((e + egran - 1) // egran) * egran

    ei = edge_index.astype(jnp.int32)
    pad = jnp.full((e_pad - e,), n, jnp.int32)
    src = jnp.concatenate([ei[0], pad]).reshape(e_pad // BLK, BLK)
    dst = jnp.concatenate([ei[1], pad]).reshape(e_pad // BLK, BLK)
    nw_pad = jnp.pad(node_weight, (0, n_pad - n))[:, None]

    deg_k = _make_deg_kernel(n_pad, e_pad)
    po, pi = deg_k(src, dst)

    col = jax.ShapeDtypeStruct((n_pad, 1), jnp.float32)
    ns, nd, qa, qb = pl.pallas_call(
        _prep_kernel,
        out_shape=[col, col, col, col],
    )(po[0][:, None], po[1][:, None], pi[0][:, None], pi[1][:, None], nw_pad)

    a0, c0, u, k0 = pl.pallas_call(
        _wprep_kernel,
        out_shape=[
            jax.ShapeDtypeStruct((1, d), jnp.float32),
            jax.ShapeDtypeStruct((1, d), jnp.float32),
            jax.ShapeDtypeStruct((d, 1), jnp.float32),
            jax.ShapeDtypeStruct((1, 1), jnp.float32),
        ],
    )(w_lin[None, :], b_lin[None, :], W0, Wout, w_pred[:, None],
      bout[None, :], b_pred[None, :])

    seg2 = _make_scalar_segsum_kernel(n_pad, e_pad, 2)
    ss, tt = seg2(qa[:, 0], qb[:, 0], src, dst)

    rblk = 1024
    nblocks = n_pad // rblk
    dh = d // 2
    colspec = pl.BlockSpec((rblk, 1), lambda i: (i, 0))
    rowspec = pl.BlockSpec((1, d), lambda i: (0, 0))
    matspec = pl.BlockSpec((rblk, d), lambda i: (i, 0))
    ysspec = pl.BlockSpec((NC, rblk, dh), lambda i: (0, i, 0))
    h0, ys = pl.pallas_call(
        _h0_kernel,
        grid=(nblocks,),
        in_specs=[colspec] * 8 + [rowspec] * 3,
        out_specs=[matspec, ysspec],
        out_shape=[jax.ShapeDtypeStruct((n_pad, d), jnp.float32),
                   jax.ShapeDtypeStruct((NC, n_pad, dh), jnp.float32)],
    )(ss[0][:, None], ss[1][:, None], tt[0][:, None], tt[1][:, None],
      qa, qb, nd, ns, a0, c0, b0[None, :])

    seg128 = _make_colsplit_segsum_kernel(n_pad, e_pad, dh)
    g = seg128(ys, src, dst)

    wspec = pl.BlockSpec((d, d), lambda i: (0, 0))
    uspec = pl.BlockSpec((d, 1), lambda i: (0, 0))
    h0spec = pl.BlockSpec((1, rblk, dh), lambda i: (0, i, 0))
    h1spec = pl.BlockSpec((1, rblk, dh), lambda i: (1, i, 0))
    z = pl.pallas_call(
        _conv1_kernel,
        grid=(nblocks,),
        in_specs=[h0spec, h1spec, h0spec, h1spec, wspec, rowspec,
                  colspec, colspec, matspec, uspec],
        out_specs=colspec,
        out_shape=col,
    )(g, g, ys, ys, W1, b1[None, :], nd, ns, h0, u)

    seg1 = _make_scalar_segsum_kernel(n_pad, e_pad, 1)
    (pp,) = seg1(z[:, 0], src, dst)

    logits = pl.pallas_call(
        _out_kernel,
        out_shape=col,
    )(pp[0][:, None], pp[1][:, None], z, nd, k0)

    return logits[:n, 0]
